# baseline (speedup 1.0000x reference)
import jax
import jax.numpy as jnp
from jax.experimental import pallas as pl
from jax.experimental.pallas import tpu as pltpu

_A = 100000
_C = 80
_B = 4
_G = 32
_BLK_A = 5000
_NBLK = _A // _BLK_A

_ALPHA = 0.25


def _body(cls_ref, reg_ref, anc_ref, ann_ref, cls_out_ref, reg_out_ref,
          acc_ref):
    j = pl.program_id(0)
    i = pl.program_id(1)

    @pl.when(i == 0)
    def _init():
        acc_ref[0, j] = 0.0
        acc_ref[1, j] = 0.0
        acc_ref[2, j] = 0.0

    anc = anc_ref[0]
    reg = reg_ref[0, 0]
    ann = ann_ref[0]

    a_y1 = anc[0:1, :]
    a_x1 = anc[1:2, :]
    a_y2 = anc[2:3, :]
    a_x2 = anc[3:4, :]

    b_x1 = ann[:, 0:1]
    b_y1 = ann[:, 1:2]
    b_x2 = ann[:, 2:3]
    b_y2 = ann[:, 3:4]
    b_cls = ann[:, 4:5]

    area_b = (b_x2 - b_x1) * (b_y2 - b_y1)
    iw = jnp.maximum(jnp.minimum(a_x2, b_x2) - jnp.maximum(a_x1, b_x1), 0.0)
    ih = jnp.maximum(jnp.minimum(a_y2, b_y2) - jnp.maximum(a_y1, b_y1), 0.0)
    area_a = (a_y2 - a_y1) * (a_x2 - a_x1)
    inter = iw * ih
    ua = jnp.maximum(area_a + area_b - inter, 1e-08)
    iou = inter / ua

    valid_f = jnp.where(b_cls != -1.0, 1.0, 0.0)
    iou = iou * valid_f + (valid_f - 1.0)

    iou_max = jnp.max(iou, axis=0, keepdims=True)
    g_iota = jax.lax.broadcasted_iota(jnp.int32, iou.shape, 0)
    iou_arg = jnp.min(jnp.where(iou == iou_max, g_iota, _G), axis=0,
                      keepdims=True)
    onehot = jnp.where(g_iota == iou_arg, 1.0, 0.0)

    assigned = jax.lax.dot_general(
        ann, onehot, (((0,), (0,)), ((), ())),
        preferred_element_type=jnp.float32)
    g_x1 = assigned[0:1, :]
    g_y1 = assigned[1:2, :]
    g_x2 = assigned[2:3, :]
    g_y2 = assigned[3:4, :]

    thr = jnp.where((g_x2 - g_x1) * (g_y2 - g_y1) > 100.0, 0.5, 0.15)
    pos_f = jnp.where(iou_max >= thr, 1.0, 0.0)
    npos = jnp.sum(pos_f)

    cls = jnp.clip(cls_ref[0], 0.0001, 1.0 - 0.0001)
    om = 1.0 - cls
    dense_sum = jnp.sum(cls * cls * (-jnp.log(om)))

    c_iota_g = jax.lax.broadcasted_iota(jnp.int32, (_G, _C), 1)
    class_oh = jnp.where(b_cls.astype(jnp.int32) == c_iota_g, 1.0, 0.0)
    cls_g = jax.lax.dot_general(
        class_oh, cls, (((1,), (1,)), ((), ())),
        preferred_element_type=jnp.float32,
        precision=jax.lax.Precision.HIGHEST)
    v = jnp.sum(cls_g * (onehot * pos_f), axis=0,
                keepdims=True)
    v = jnp.clip(v, 0.0001, 1.0 - 0.0001)
    ov = 1.0 - v
    corr = pos_f * (_ALPHA * ov * ov * (-jnp.log(v))
                    - (1.0 - _ALPHA) * v * v * (-jnp.log(ov)))
    cls_part = (1.0 - _ALPHA) * dense_sum + jnp.sum(corr)

    aw = a_x2 - a_x1
    ah = a_y2 - a_y1
    acx = a_x1 + 0.5 * aw
    acy = a_y1 + 0.5 * ah
    gw = g_x2 - g_x1
    gh = g_y2 - g_y1
    gcx = g_x1 + 0.5 * gw
    gcy = g_y1 + 0.5 * gh
    gw = jnp.maximum(gw, 1.0)
    gh = jnp.maximum(gh, 1.0)
    t0 = (gcy - acy) / ah
    t1 = (gcx - acx) / aw
    t2 = jnp.log(gh / ah)
    t3 = jnp.log(gw / aw)

    reg_part = 0.0
    for k, tk in enumerate((t0, t1, t2, t3)):
        diff = jnp.abs(tk - reg[k:k + 1, :])
        rl = jnp.where(diff <= 1.0 / 9.0, 0.5 * 9.0 * diff * diff,
                       diff - 0.5 / 9.0)
        reg_part = reg_part + jnp.sum(rl * pos_f)

    acc_ref[0, j] += cls_part
    acc_ref[1, j] += reg_part
    acc_ref[2, j] += npos

    @pl.when(jnp.logical_and(j == _B - 1, i == _NBLK - 1))
    def _finalize():
        cls_total = 0.0
        reg_total = 0.0
        for jj in range(_B):
            np_j = acc_ref[2, jj]
            cls_total += acc_ref[0, jj] / jnp.maximum(np_j, 1.0)
            reg_total += acc_ref[1, jj] / jnp.maximum(np_j * 4.0, 1.0)
        cls_out_ref[0] = cls_total / _B
        reg_out_ref[0] = reg_total * (50.0 / _B)


def kernel(classifications, regressions, anchors, annotations):
    anc_t = jnp.transpose(
        anchors[0].reshape(_NBLK, _BLK_A, 4), (0, 2, 1))
    reg_t = jnp.transpose(
        regressions.reshape(_B, _NBLK, _BLK_A, 4),
        (0, 1, 3, 2))
    cls_loss, reg_loss = pl.pallas_call(
        _body,
        grid=(_B, _NBLK),
        in_specs=[
            pl.BlockSpec((1, _BLK_A, _C), lambda j, i: (j, i, 0)),
            pl.BlockSpec((1, 1, 4, _BLK_A), lambda j, i: (j, i, 0, 0)),
            pl.BlockSpec((1, 4, _BLK_A), lambda j, i: (i, 0, 0)),
            pl.BlockSpec((1, _G, 5), lambda j, i: (j, 0, 0)),
        ],
        out_specs=[
            pl.BlockSpec(memory_space=pltpu.SMEM),
            pl.BlockSpec(memory_space=pltpu.SMEM),
        ],
        out_shape=[
            jax.ShapeDtypeStruct((1,), jnp.float32),
            jax.ShapeDtypeStruct((1,), jnp.float32),
        ],
        scratch_shapes=[pltpu.SMEM((3, _B), jnp.float32)],
    )(classifications, reg_t, anc_t, annotations)
    return (cls_loss, reg_loss)


# speedup vs baseline: 1.2651x; 1.2651x over previous
import jax
import jax.numpy as jnp
from jax.experimental import pallas as pl
from jax.experimental.pallas import tpu as pltpu

_A = 100000
_C = 80
_B = 4
_G = 32
_BLK_A = 5000
_NBLK = _A // _BLK_A

_ALPHA = 0.25


def _body(cls_ref, reg_ref, anc_ref, ann_ref, cls_out_ref, reg_out_ref,
          acc_ref):
    j = pl.program_id(0)
    i = pl.program_id(1)

    @pl.when(i == 0)
    def _init():
        acc_ref[0, j] = 0.0
        acc_ref[1, j] = 0.0
        acc_ref[2, j] = 0.0

    anc = anc_ref[0]
    reg = reg_ref[0, 0]
    ann = ann_ref[0]

    a_y1 = anc[0:1, :]
    a_x1 = anc[1:2, :]
    a_y2 = anc[2:3, :]
    a_x2 = anc[3:4, :]

    b_x1 = ann[:, 0:1]
    b_y1 = ann[:, 1:2]
    b_x2 = ann[:, 2:3]
    b_y2 = ann[:, 3:4]
    b_cls = ann[:, 4:5]

    area_b = (b_x2 - b_x1) * (b_y2 - b_y1)
    iw = jnp.maximum(jnp.minimum(a_x2, b_x2) - jnp.maximum(a_x1, b_x1), 0.0)
    ih = jnp.maximum(jnp.minimum(a_y2, b_y2) - jnp.maximum(a_y1, b_y1), 0.0)
    area_a = (a_y2 - a_y1) * (a_x2 - a_x1)
    inter = iw * ih
    ua = jnp.maximum(area_a + area_b - inter, 1e-08)
    iou = inter / ua

    valid_f = jnp.where(b_cls != -1.0, 1.0, 0.0)
    iou = iou * valid_f + (valid_f - 1.0)

    iou_max = jnp.max(iou, axis=0, keepdims=True)
    g_iota = jax.lax.broadcasted_iota(jnp.int32, iou.shape, 0)
    iou_arg = jnp.min(jnp.where(iou == iou_max, g_iota, _G), axis=0,
                      keepdims=True)
    onehot = jnp.where(g_iota == iou_arg, 1.0, 0.0)

    assigned = jax.lax.dot_general(
        ann, onehot, (((0,), (0,)), ((), ())),
        preferred_element_type=jnp.float32)
    g_x1 = assigned[0:1, :]
    g_y1 = assigned[1:2, :]
    g_x2 = assigned[2:3, :]
    g_y2 = assigned[3:4, :]

    thr = jnp.where((g_x2 - g_x1) * (g_y2 - g_y1) > 100.0, 0.5, 0.15)
    pos_f = jnp.where(iou_max >= thr, 1.0, 0.0)
    npos = jnp.sum(pos_f)

    cls = jnp.clip(cls_ref[0], 0.0001, 1.0 - 0.0001)
    om = 1.0 - cls
    dense_sum = jnp.sum(cls * cls * (-jnp.log(om)))

    c_iota_g = jax.lax.broadcasted_iota(jnp.int32, (_G, _C), 1)
    class_oh = jnp.where(b_cls.astype(jnp.int32) == c_iota_g, 1.0, 0.0)
    cls_g = jax.lax.dot_general(
        class_oh, cls, (((1,), (1,)), ((), ())),
        preferred_element_type=jnp.float32)
    v = jnp.sum(cls_g * (onehot * pos_f), axis=0,
                keepdims=True)
    v = jnp.clip(v, 0.0001, 1.0 - 0.0001)
    ov = 1.0 - v
    corr = pos_f * (_ALPHA * ov * ov * (-jnp.log(v))
                    - (1.0 - _ALPHA) * v * v * (-jnp.log(ov)))
    cls_part = (1.0 - _ALPHA) * dense_sum + jnp.sum(corr)

    aw = a_x2 - a_x1
    ah = a_y2 - a_y1
    acx = a_x1 + 0.5 * aw
    acy = a_y1 + 0.5 * ah
    gw = g_x2 - g_x1
    gh = g_y2 - g_y1
    gcx = g_x1 + 0.5 * gw
    gcy = g_y1 + 0.5 * gh
    gw = jnp.maximum(gw, 1.0)
    gh = jnp.maximum(gh, 1.0)
    t0 = (gcy - acy) / ah
    t1 = (gcx - acx) / aw
    t2 = jnp.log(gh / ah)
    t3 = jnp.log(gw / aw)

    reg_part = 0.0
    for k, tk in enumerate((t0, t1, t2, t3)):
        diff = jnp.abs(tk - reg[k:k + 1, :])
        rl = jnp.where(diff <= 1.0 / 9.0, 0.5 * 9.0 * diff * diff,
                       diff - 0.5 / 9.0)
        reg_part = reg_part + jnp.sum(rl * pos_f)

    acc_ref[0, j] += cls_part
    acc_ref[1, j] += reg_part
    acc_ref[2, j] += npos

    @pl.when(jnp.logical_and(j == _B - 1, i == _NBLK - 1))
    def _finalize():
        cls_total = 0.0
        reg_total = 0.0
        for jj in range(_B):
            np_j = acc_ref[2, jj]
            cls_total += acc_ref[0, jj] / jnp.maximum(np_j, 1.0)
            reg_total += acc_ref[1, jj] / jnp.maximum(np_j * 4.0, 1.0)
        cls_out_ref[0] = cls_total / _B
        reg_out_ref[0] = reg_total * (50.0 / _B)


def kernel(classifications, regressions, anchors, annotations):
    anc_t = jnp.transpose(
        anchors[0].reshape(_NBLK, _BLK_A, 4), (0, 2, 1))
    reg_t = jnp.transpose(
        regressions.reshape(_B, _NBLK, _BLK_A, 4),
        (0, 1, 3, 2))
    cls_loss, reg_loss = pl.pallas_call(
        _body,
        grid=(_B, _NBLK),
        in_specs=[
            pl.BlockSpec((1, _BLK_A, _C), lambda j, i: (j, i, 0)),
            pl.BlockSpec((1, 1, 4, _BLK_A), lambda j, i: (j, i, 0, 0)),
            pl.BlockSpec((1, 4, _BLK_A), lambda j, i: (i, 0, 0)),
            pl.BlockSpec((1, _G, 5), lambda j, i: (j, 0, 0)),
        ],
        out_specs=[
            pl.BlockSpec(memory_space=pltpu.SMEM),
            pl.BlockSpec(memory_space=pltpu.SMEM),
        ],
        out_shape=[
            jax.ShapeDtypeStruct((1,), jnp.float32),
            jax.ShapeDtypeStruct((1,), jnp.float32),
        ],
        scratch_shapes=[pltpu.SMEM((3, _B), jnp.float32)],
    )(classifications, reg_t, anc_t, annotations)
    return (cls_loss, reg_loss)
